# Initial kernel scaffold; baseline (speedup 1.0000x reference)
import jax
import jax.numpy as jnp
from jax.experimental import pallas as pl


def kernel(x, rbf, dists, edge_index, batch, W_n2m, W_r2m, W_lin1, W_lin2, b_lin2, W_m2n):
    raise NotImplementedError("write your pallas kernel here")


# speedup vs baseline: 1.0717x; 1.0717x over previous
import functools

import jax
import jax.numpy as jnp
from jax import lax
from jax.experimental import pallas as pl
from jax.experimental.pallas import tpu as pltpu
from jax.experimental.pallas import tpu_sc as plsc

N_NODES = 10000
N_EDGES = 160000
HIDDEN = 256
RADIAL = 64
FH = 128
CUTOFF = 5.0

NC = 2
NS = 16
EPT = N_EDGES // NS
K = 80
NCHUNK = EPT // K
NG = 5
GC = NCHUNK // NG
NPAD = 10240
NPT = NPAD // NS
ZR = 128


def _wfold_body(wn2m, wlin1, wlin2, b2, wm2n, wc1, wc2, bvec):
    f32 = jnp.float32
    hi = jax.lax.Precision.HIGHEST
    wc1[...] = jnp.dot(wn2m[...], wlin1[...], precision=hi, preferred_element_type=f32)
    wc2[...] = jnp.dot(wlin2[...], wm2n[...], precision=hi, preferred_element_type=f32)
    bvec[...] = jnp.dot(b2[...], wm2n[...], precision=hi, preferred_element_type=f32)


def _h1_body(x, wc1, h1a, h1b):
    h = jnp.dot(x[...], wc1[...], precision=jax.lax.Precision.HIGHEST,
                preferred_element_type=jnp.float32)
    h1a[...] = h[:, :FH]
    h1b[...] = h[:, FH:]


def _edge_body(rbf, d2, wr2m, wea, web):
    c = 0.5 * (jnp.cos(d2[...] * (jnp.pi / CUTOFF)) + 1.0)
    w = jnp.dot(rbf[...] * c, wr2m[...], precision=jax.lax.Precision.HIGHEST,
                preferred_element_type=jnp.float32)
    wea[...] = w[:, :FH]
    web[...] = w[:, FH:]


def _out_body(agga, aggb, wc2, bvec, out):
    hi = jax.lax.Precision.HIGHEST
    acc = jnp.dot(agga[...], wc2[:FH, :], precision=hi, preferred_element_type=jnp.float32)
    acc += jnp.dot(aggb[...], wc2[FH:, :], precision=hi, preferred_element_type=jnp.float32)
    out[...] = acc + bvec[...]


def _sc_half(h1_hbm, we_hbm, src4, dst4, out_hbm, sid, agg_s, srcbuf, dstbuf,
             rows, wbuf, zer_hbm, sem):
    for z in range(NPT // ZR):
        pltpu.sync_copy(zer_hbm, agg_s.at[pl.ds(sid * NPT + z * ZR, ZR)])
    plsc.subcore_barrier()

    def group(g, carry):
        pltpu.sync_copy(src4.at[sid, g], srcbuf)
        pltpu.sync_copy(dst4.at[sid, g], dstbuf)

        def chunk(j, c1):
            pltpu.async_copy(h1_hbm.at[srcbuf.at[j]], rows, sem).wait()
            e0 = sid * EPT + (g * GC + j) * K
            pltpu.sync_copy(we_hbm.at[pl.ds(e0, K)], wbuf)

            def row(r, c2):
                for cc in range(FH // 16):
                    s = pl.ds(cc * 16, 16)
                    rows[r, s] = rows[r, s] * wbuf[r, s]
                return c2
            lax.fori_loop(0, K, row, 0, unroll=2)

            pltpu.sync_copy(rows, agg_s.at[dstbuf.at[j]], add=True)
            return c1

        lax.fori_loop(0, GC, chunk, 0)
        return carry

    lax.fori_loop(0, NG, group, 0)
    plsc.subcore_barrier()

    for z in range(NPT // ZR):
        o = sid * NPT + z * ZR
        pltpu.sync_copy(agg_s.at[pl.ds(o, ZR)], out_hbm.at[pl.ds(o, ZR)])


def _sc_body(h1a, h1b, wea, web, src4, dst4, zer, agga, aggb,
             agg_s, srcbuf, dstbuf, rows, wbuf, sem):
    cid = lax.axis_index("c")
    sid = lax.axis_index("s")

    @pl.when(cid == 0)
    def _():
        _sc_half(h1a, wea, src4, dst4, agga, sid, agg_s, srcbuf, dstbuf,
                 rows, wbuf, zer, sem)

    @pl.when(cid == 1)
    def _():
        _sc_half(h1b, web, src4, dst4, aggb, sid, agg_s, srcbuf, dstbuf,
                 rows, wbuf, zer, sem)


def kernel(x, rbf, dists, edge_index, batch, W_n2m, W_r2m, W_lin1, W_lin2,
           b_lin2, W_m2n):
    f32 = jnp.float32
    sds = jax.ShapeDtypeStruct

    wc1, wc2, bvec = pl.pallas_call(
        _wfold_body,
        out_shape=(sds((HIDDEN, HIDDEN), f32), sds((HIDDEN, HIDDEN), f32),
                   sds((1, HIDDEN), f32)),
    )(W_n2m, W_lin1, W_lin2, b_lin2.reshape(1, HIDDEN), W_m2n)

    NB = 5
    h1a, h1b = pl.pallas_call(
        _h1_body,
        grid=(NB,),
        in_specs=[pl.BlockSpec((N_NODES // NB, HIDDEN), lambda i: (i, 0)),
                  pl.BlockSpec((HIDDEN, HIDDEN), lambda i: (0, 0))],
        out_specs=[pl.BlockSpec((N_NODES // NB, FH), lambda i: (i, 0)),
                   pl.BlockSpec((N_NODES // NB, FH), lambda i: (i, 0))],
        out_shape=(sds((N_NODES, FH), f32), sds((N_NODES, FH), f32)),
    )(x, wc1)

    EB = 32
    wea, web = pl.pallas_call(
        _edge_body,
        grid=(EB,),
        in_specs=[pl.BlockSpec((N_EDGES // EB, RADIAL), lambda i: (i, 0)),
                  pl.BlockSpec((N_EDGES // EB, 1), lambda i: (i, 0)),
                  pl.BlockSpec((RADIAL, HIDDEN), lambda i: (0, 0))],
        out_specs=[pl.BlockSpec((N_EDGES // EB, FH), lambda i: (i, 0)),
                   pl.BlockSpec((N_EDGES // EB, FH), lambda i: (i, 0))],
        out_shape=(sds((N_EDGES, FH), f32), sds((N_EDGES, FH), f32)),
    )(rbf, dists.reshape(N_EDGES, 1), W_r2m)

    src4 = edge_index[0].astype(jnp.int32).reshape(NS, NG, GC, K)
    dst4 = edge_index[1].astype(jnp.int32).reshape(NS, NG, GC, K)
    zer = jnp.zeros((ZR, FH), f32)

    mesh = plsc.VectorSubcoreMesh(core_axis_name="c", subcore_axis_name="s",
                                  num_cores=NC, num_subcores=NS)
    agga, aggb = pl.kernel(
        _sc_body,
        out_type=(sds((NPAD, FH), f32), sds((NPAD, FH), f32)),
        mesh=mesh,
        scratch_types=[
            pltpu.VMEM_SHARED((NPAD, FH), f32),
            pltpu.VMEM((GC, K), jnp.int32),
            pltpu.VMEM((GC, K), jnp.int32),
            pltpu.VMEM((K, FH), f32),
            pltpu.VMEM((K, FH), f32),
            pltpu.SemaphoreType.DMA,
        ],
    )(h1a, h1b, wea, web, src4, dst4, zer)

    out = pl.pallas_call(
        _out_body,
        grid=(NB,),
        in_specs=[pl.BlockSpec((N_NODES // NB, FH), lambda i: (i, 0)),
                  pl.BlockSpec((N_NODES // NB, FH), lambda i: (i, 0)),
                  pl.BlockSpec((HIDDEN, HIDDEN), lambda i: (0, 0)),
                  pl.BlockSpec((1, HIDDEN), lambda i: (0, 0))],
        out_specs=pl.BlockSpec((N_NODES // NB, HIDDEN), lambda i: (i, 0)),
        out_shape=sds((N_NODES, HIDDEN), f32),
    )(agga, aggb, wc2, bvec)
    return out


# speedup vs baseline: 1.4148x; 1.3200x over previous
import functools

import jax
import jax.numpy as jnp
from jax import lax
from jax.experimental import pallas as pl
from jax.experimental.pallas import tpu as pltpu
from jax.experimental.pallas import tpu_sc as plsc

N_NODES = 10000
N_EDGES = 160000
HIDDEN = 256
RADIAL = 64
FH = 128
CUTOFF = 5.0

NC = 2
NS = 16
EPT = N_EDGES // NS
K = 40
NCHUNK = EPT // K
NG = 25
GC = NCHUNK // NG
NPAD = 10240
NPT = NPAD // NS
ZR = 128


def _wfold_body(wn2m, wlin1, wlin2, b2, wm2n, wc1, wc2, bvec):
    f32 = jnp.float32
    wc1[...] = jnp.dot(wn2m[...], wlin1[...], preferred_element_type=f32)
    wc2[...] = jnp.dot(wlin2[...], wm2n[...], preferred_element_type=f32)
    bvec[...] = jnp.dot(b2[...], wm2n[...], preferred_element_type=f32)


def _h1_body(x, wc1, h1a, h1b):
    h = jnp.dot(x[...], wc1[...],
                preferred_element_type=jnp.float32)
    h1a[...] = h[:, :FH]
    h1b[...] = h[:, FH:]


def _edge_body(rbf, d2, wr2m, wea, web):
    c = 0.5 * (jnp.cos(d2[...] * (jnp.pi / CUTOFF)) + 1.0)
    w = jnp.dot(rbf[...] * c, wr2m[...],
                preferred_element_type=jnp.float32)
    wea[...] = w[:, :FH]
    web[...] = w[:, FH:]


def _out_body(agga, aggb, wc2, bvec, out):
    acc = jnp.dot(agga[...], wc2[:FH, :], preferred_element_type=jnp.float32)
    acc += jnp.dot(aggb[...], wc2[FH:, :], preferred_element_type=jnp.float32)
    out[...] = acc + bvec[...]


def _mul_rows(rows, wbuf):
    def row(r, c2):
        for cc in range(FH // 16):
            s = pl.ds(cc * 16, 16)
            rows[r, s] = rows[r, s] * wbuf[r, s]
        return c2
    lax.fori_loop(0, K, row, 0, unroll=2)


def _sc_half(h1_hbm, we_hbm, src4, dst4, out_hbm, sid, agg_s, srcbuf, dstbuf,
             rows0, wbuf0, rows1, wbuf1, zer_hbm, sem0, sem1):
    for z in range(NPT // ZR):
        pltpu.sync_copy(zer_hbm, agg_s.at[pl.ds(sid * NPT + z * ZR, ZR)])
    plsc.subcore_barrier()

    rw = ((rows0, wbuf0, sem0), (rows1, wbuf1, sem1))

    def issue(g, j, rows, wbuf, sem):
        e0 = sid * EPT + g * (GC * K) + j * K
        pltpu.async_copy(h1_hbm.at[srcbuf.at[j]], rows, sem)
        pltpu.async_copy(we_hbm.at[pl.ds(e0, K)], wbuf, sem)

    def drain(rows, wbuf, sem):
        pltpu.make_async_copy(we_hbm.at[pl.ds(0, K)], rows, sem).wait()
        pltpu.make_async_copy(we_hbm.at[pl.ds(0, K)], wbuf, sem).wait()

    def group(g, carry):
        pltpu.sync_copy(src4.at[sid, g], srcbuf)
        pltpu.sync_copy(dst4.at[sid, g], dstbuf)
        issue(g, 0, *rw[0])
        for j in range(GC):
            rows, wbuf, sem = rw[j % 2]
            nrows, nwbuf, nsem = rw[(j + 1) % 2]
            if j + 1 < GC:
                issue(g, j + 1, nrows, nwbuf, nsem)
            drain(rows, wbuf, sem)
            _mul_rows(rows, wbuf)
            pltpu.sync_copy(rows, agg_s.at[dstbuf.at[j]], add=True)
        return carry

    lax.fori_loop(0, NG, group, 0)
    plsc.subcore_barrier()

    for z in range(NPT // ZR):
        o = sid * NPT + z * ZR
        pltpu.sync_copy(agg_s.at[pl.ds(o, ZR)], out_hbm.at[pl.ds(o, ZR)])


def _sc_body(h1a, h1b, wea, web, src4, dst4, zer, agga, aggb,
             agg_s, srcbuf, dstbuf, rows0, wbuf0, rows1, wbuf1, sem0, sem1):
    cid = lax.axis_index("c")
    sid = lax.axis_index("s")

    @pl.when(cid == 0)
    def _():
        _sc_half(h1a, wea, src4, dst4, agga, sid, agg_s, srcbuf, dstbuf,
                 rows0, wbuf0, rows1, wbuf1, zer, sem0, sem1)

    @pl.when(cid == 1)
    def _():
        _sc_half(h1b, web, src4, dst4, aggb, sid, agg_s, srcbuf, dstbuf,
                 rows0, wbuf0, rows1, wbuf1, zer, sem0, sem1)


def kernel(x, rbf, dists, edge_index, batch, W_n2m, W_r2m, W_lin1, W_lin2,
           b_lin2, W_m2n):
    f32 = jnp.float32
    sds = jax.ShapeDtypeStruct

    wc1, wc2, bvec = pl.pallas_call(
        _wfold_body,
        out_shape=(sds((HIDDEN, HIDDEN), f32), sds((HIDDEN, HIDDEN), f32),
                   sds((1, HIDDEN), f32)),
    )(W_n2m, W_lin1, W_lin2, b_lin2.reshape(1, HIDDEN), W_m2n)

    NB = 5
    h1a, h1b = pl.pallas_call(
        _h1_body,
        grid=(NB,),
        in_specs=[pl.BlockSpec((N_NODES // NB, HIDDEN), lambda i: (i, 0)),
                  pl.BlockSpec((HIDDEN, HIDDEN), lambda i: (0, 0))],
        out_specs=[pl.BlockSpec((N_NODES // NB, FH), lambda i: (i, 0)),
                   pl.BlockSpec((N_NODES // NB, FH), lambda i: (i, 0))],
        out_shape=(sds((N_NODES, FH), f32), sds((N_NODES, FH), f32)),
    )(x, wc1)

    EB = 32
    wea, web = pl.pallas_call(
        _edge_body,
        grid=(EB,),
        in_specs=[pl.BlockSpec((N_EDGES // EB, RADIAL), lambda i: (i, 0)),
                  pl.BlockSpec((N_EDGES // EB, 1), lambda i: (i, 0)),
                  pl.BlockSpec((RADIAL, HIDDEN), lambda i: (0, 0))],
        out_specs=[pl.BlockSpec((N_EDGES // EB, FH), lambda i: (i, 0)),
                   pl.BlockSpec((N_EDGES // EB, FH), lambda i: (i, 0))],
        out_shape=(sds((N_EDGES, FH), f32), sds((N_EDGES, FH), f32)),
    )(rbf, dists.reshape(N_EDGES, 1), W_r2m)

    src4 = edge_index[0].astype(jnp.int32).reshape(NS, NG, GC, K)
    dst4 = edge_index[1].astype(jnp.int32).reshape(NS, NG, GC, K)
    zer = jnp.zeros((ZR, FH), f32)

    mesh = plsc.VectorSubcoreMesh(core_axis_name="c", subcore_axis_name="s",
                                  num_cores=NC, num_subcores=NS)
    agga, aggb = pl.kernel(
        _sc_body,
        out_type=(sds((NPAD, FH), f32), sds((NPAD, FH), f32)),
        mesh=mesh,
        scratch_types=[
            pltpu.VMEM_SHARED((NPAD, FH), f32),
            pltpu.VMEM((GC, K), jnp.int32),
            pltpu.VMEM((GC, K), jnp.int32),
            pltpu.VMEM((K, FH), f32),
            pltpu.VMEM((K, FH), f32),
            pltpu.VMEM((K, FH), f32),
            pltpu.VMEM((K, FH), f32),
            pltpu.SemaphoreType.DMA,
            pltpu.SemaphoreType.DMA,
        ],
    )(h1a, h1b, wea, web, src4, dst4, zer)

    out = pl.pallas_call(
        _out_body,
        grid=(NB,),
        in_specs=[pl.BlockSpec((N_NODES // NB, FH), lambda i: (i, 0)),
                  pl.BlockSpec((N_NODES // NB, FH), lambda i: (i, 0)),
                  pl.BlockSpec((HIDDEN, HIDDEN), lambda i: (0, 0)),
                  pl.BlockSpec((1, HIDDEN), lambda i: (0, 0))],
        out_specs=pl.BlockSpec((N_NODES // NB, HIDDEN), lambda i: (i, 0)),
        out_shape=sds((N_NODES, HIDDEN), f32),
    )(agga, aggb, wc2, bvec)
    return out


# speedup vs baseline: 1.7853x; 1.2619x over previous
import functools

import jax
import jax.numpy as jnp
from jax import lax
from jax.experimental import pallas as pl
from jax.experimental.pallas import tpu as pltpu
from jax.experimental.pallas import tpu_sc as plsc

N_NODES = 10000
N_EDGES = 160000
HIDDEN = 256
RADIAL = 64
FH = 128
CUTOFF = 5.0

NC = 2
NS = 16
EPT = N_EDGES // NS
K = 40
NCHUNK = EPT // K
NG = 25
GC = NCHUNK // NG
NPAD = 10240
NPT = NPAD // NS
ZR = 128


def _wfold_body(wn2m, wlin1, wlin2, b2, wm2n, wc1, wc2, bvec):
    f32 = jnp.float32
    wc1[...] = jnp.dot(wn2m[...], wlin1[...], preferred_element_type=f32)
    wc2[...] = jnp.dot(wlin2[...], wm2n[...], preferred_element_type=f32)
    bvec[...] = jnp.dot(b2[...], wm2n[...], preferred_element_type=f32)


def _h1_body(x, wc1, h1a, h1b):
    h = jnp.dot(x[...], wc1[...],
                preferred_element_type=jnp.float32)
    h1a[...] = h[:, :FH]
    h1b[...] = h[:, FH:]


def _cos_body(d2, c2):
    c2[...] = 0.5 * (jnp.cos(d2[...] * (jnp.pi / CUTOFF)) + 1.0)


def _edge_body(rbf, c, wr2m, wea, web):
    w = jnp.dot(rbf[...] * c[...], wr2m[...],
                preferred_element_type=jnp.float32)
    wea[...] = w[:, :FH]
    web[...] = w[:, FH:]


def _out_body(agga, aggb, wc2, bvec, out):
    acc = jnp.dot(agga[...], wc2[:FH, :], preferred_element_type=jnp.float32)
    acc += jnp.dot(aggb[...], wc2[FH:, :], preferred_element_type=jnp.float32)
    out[...] = acc + bvec[...]


def _mul_rows(rows, wbuf):
    def row(r, c2):
        for cc in range(FH // 16):
            s = pl.ds(cc * 16, 16)
            rows[r, s] = rows[r, s] * wbuf[r, s]
        return c2
    lax.fori_loop(0, K, row, 0, unroll=2)


def _sc_half(h1_hbm, we_hbm, src4, dst4, out_hbm, sid, agg_s, srcbuf, dstbuf,
             rows0, wbuf0, rows1, wbuf1, zer_hbm, sem0, sem1):
    for z in range(NPT // ZR):
        pltpu.sync_copy(zer_hbm, agg_s.at[pl.ds(sid * NPT + z * ZR, ZR)])
    plsc.subcore_barrier()

    rw = ((rows0, wbuf0, sem0), (rows1, wbuf1, sem1))

    def issue(g, j, rows, wbuf, sem):
        e0 = sid * EPT + g * (GC * K) + j * K
        pltpu.async_copy(h1_hbm.at[srcbuf.at[j]], rows, sem)
        pltpu.async_copy(we_hbm.at[pl.ds(e0, K)], wbuf, sem)

    def drain(rows, wbuf, sem):
        pltpu.make_async_copy(we_hbm.at[pl.ds(0, K)], rows, sem).wait()
        pltpu.make_async_copy(we_hbm.at[pl.ds(0, K)], wbuf, sem).wait()

    def group(g, carry):
        pltpu.sync_copy(src4.at[sid, g], srcbuf)
        pltpu.sync_copy(dst4.at[sid, g], dstbuf)
        issue(g, 0, *rw[0])
        for j in range(GC):
            rows, wbuf, sem = rw[j % 2]
            nrows, nwbuf, nsem = rw[(j + 1) % 2]
            if j + 1 < GC:
                issue(g, j + 1, nrows, nwbuf, nsem)
            drain(rows, wbuf, sem)
            _mul_rows(rows, wbuf)
            pltpu.sync_copy(rows, agg_s.at[dstbuf.at[j]], add=True)
        return carry

    lax.fori_loop(0, NG, group, 0)
    plsc.subcore_barrier()

    for z in range(NPT // ZR):
        o = sid * NPT + z * ZR
        pltpu.sync_copy(agg_s.at[pl.ds(o, ZR)], out_hbm.at[pl.ds(o, ZR)])


def _sc_body(h1a, h1b, wea, web, src4, dst4, zer, agga, aggb,
             agg_s, srcbuf, dstbuf, rows0, wbuf0, rows1, wbuf1, sem0, sem1):
    cid = lax.axis_index("c")
    sid = lax.axis_index("s")

    @pl.when(cid == 0)
    def _():
        _sc_half(h1a, wea, src4, dst4, agga, sid, agg_s, srcbuf, dstbuf,
                 rows0, wbuf0, rows1, wbuf1, zer, sem0, sem1)

    @pl.when(cid == 1)
    def _():
        _sc_half(h1b, web, src4, dst4, aggb, sid, agg_s, srcbuf, dstbuf,
                 rows0, wbuf0, rows1, wbuf1, zer, sem0, sem1)


def kernel(x, rbf, dists, edge_index, batch, W_n2m, W_r2m, W_lin1, W_lin2,
           b_lin2, W_m2n):
    f32 = jnp.float32
    sds = jax.ShapeDtypeStruct

    wc1, wc2, bvec = pl.pallas_call(
        _wfold_body,
        out_shape=(sds((HIDDEN, HIDDEN), f32), sds((HIDDEN, HIDDEN), f32),
                   sds((1, HIDDEN), f32)),
    )(W_n2m, W_lin1, W_lin2, b_lin2.reshape(1, HIDDEN), W_m2n)

    NB = 5
    h1a, h1b = pl.pallas_call(
        _h1_body,
        grid=(NB,),
        in_specs=[pl.BlockSpec((N_NODES // NB, HIDDEN), lambda i: (i, 0)),
                  pl.BlockSpec((HIDDEN, HIDDEN), lambda i: (0, 0))],
        out_specs=[pl.BlockSpec((N_NODES // NB, FH), lambda i: (i, 0)),
                   pl.BlockSpec((N_NODES // NB, FH), lambda i: (i, 0))],
        out_shape=(sds((N_NODES, FH), f32), sds((N_NODES, FH), f32)),
    )(x, wc1)

    cden = pl.pallas_call(
        _cos_body,
        out_shape=sds((N_EDGES // 128, 128), f32),
    )(dists.reshape(N_EDGES // 128, 128))

    EB = 32
    wea, web = pl.pallas_call(
        _edge_body,
        grid=(EB,),
        in_specs=[pl.BlockSpec((N_EDGES // EB, RADIAL), lambda i: (i, 0)),
                  pl.BlockSpec((N_EDGES // EB, 1), lambda i: (i, 0)),
                  pl.BlockSpec((RADIAL, HIDDEN), lambda i: (0, 0))],
        out_specs=[pl.BlockSpec((N_EDGES // EB, FH), lambda i: (i, 0)),
                   pl.BlockSpec((N_EDGES // EB, FH), lambda i: (i, 0))],
        out_shape=(sds((N_EDGES, FH), f32), sds((N_EDGES, FH), f32)),
    )(rbf, cden.reshape(N_EDGES, 1), W_r2m)

    src4 = edge_index[0].astype(jnp.int32).reshape(NS, NG, GC, K)
    dst4 = edge_index[1].astype(jnp.int32).reshape(NS, NG, GC, K)
    zer = jnp.zeros((ZR, FH), f32)

    mesh = plsc.VectorSubcoreMesh(core_axis_name="c", subcore_axis_name="s",
                                  num_cores=NC, num_subcores=NS)
    agga, aggb = pl.kernel(
        _sc_body,
        out_type=(sds((NPAD, FH), f32), sds((NPAD, FH), f32)),
        mesh=mesh,
        scratch_types=[
            pltpu.VMEM_SHARED((NPAD, FH), f32),
            pltpu.VMEM((GC, K), jnp.int32),
            pltpu.VMEM((GC, K), jnp.int32),
            pltpu.VMEM((K, FH), f32),
            pltpu.VMEM((K, FH), f32),
            pltpu.VMEM((K, FH), f32),
            pltpu.VMEM((K, FH), f32),
            pltpu.SemaphoreType.DMA,
            pltpu.SemaphoreType.DMA,
        ],
    )(h1a, h1b, wea, web, src4, dst4, zer)

    out = pl.pallas_call(
        _out_body,
        grid=(NB,),
        in_specs=[pl.BlockSpec((N_NODES // NB, FH), lambda i: (i, 0)),
                  pl.BlockSpec((N_NODES // NB, FH), lambda i: (i, 0)),
                  pl.BlockSpec((HIDDEN, HIDDEN), lambda i: (0, 0)),
                  pl.BlockSpec((1, HIDDEN), lambda i: (0, 0))],
        out_specs=pl.BlockSpec((N_NODES // NB, HIDDEN), lambda i: (i, 0)),
        out_shape=sds((N_NODES, HIDDEN), f32),
    )(agga, aggb, wc2, bvec)
    return out


# speedup vs baseline: 2.5002x; 1.4004x over previous
import functools

import jax
import jax.numpy as jnp
from jax import lax
from jax.experimental import pallas as pl
from jax.experimental.pallas import tpu as pltpu
from jax.experimental.pallas import tpu_sc as plsc

N_NODES = 10000
N_EDGES = 160000
HIDDEN = 256
RADIAL = 64
FH = 128
CUTOFF = 5.0

NC = 2
NS = 16
EPT = N_EDGES // NS
K = 40
NCHUNK = EPT // K
NG = 25
GC = NCHUNK // NG
NPAD = 10240
NPT = NPAD // NS
ZR = 128


def _wfold_body(wn2m, wlin1, wlin2, b2, wm2n, wc1, wc2, bvec):
    f32 = jnp.float32
    wc1[...] = jnp.dot(wn2m[...], wlin1[...], preferred_element_type=f32)
    wc2[...] = jnp.dot(wlin2[...], wm2n[...], preferred_element_type=f32)
    bvec[...] = jnp.dot(b2[...], wm2n[...], preferred_element_type=f32)


def _h1_body(x, wc1, h1a, h1b):
    h = jnp.dot(x[...], wc1[...],
                preferred_element_type=jnp.float32)
    h1a[...] = h[:, :FH]
    h1b[...] = h[:, FH:]


def _cos_body(d2, c2):
    c2[...] = 0.5 * (jnp.cos(d2[...] * (jnp.pi / CUTOFF)) + 1.0)


def _edge_body(rbf, c, wr2m, wea, web):
    w = jnp.dot(rbf[...] * c[...], wr2m[...],
                preferred_element_type=jnp.float32)
    wea[...] = w[:, :FH]
    web[...] = w[:, FH:]


def _out_body(agga, aggb, wc2, bvec, out):
    acc = jnp.dot(agga[...], wc2[:FH, :], preferred_element_type=jnp.float32)
    acc += jnp.dot(aggb[...], wc2[FH:, :], preferred_element_type=jnp.float32)
    out[...] = acc + bvec[...]


def _mul_rows(rows, wbuf):
    @plsc.parallel_loop(0, K, 1, unroll=4)
    def _(r):
        for cc in range(FH // 16):
            s = pl.ds(cc * 16, 16)
            rows[r, s] = rows[r, s] * wbuf[r, s]


def _sc_half(h1_hbm, we_hbm, src4, dst4, out_hbm, sid, agg_s, srcbuf, dstbuf,
             rows0, wbuf0, rows1, wbuf1, zer_hbm, sem0, sem1):
    for z in range(NPT // ZR):
        pltpu.sync_copy(zer_hbm, agg_s.at[pl.ds(sid * NPT + z * ZR, ZR)])
    plsc.subcore_barrier()

    rw = ((rows0, wbuf0, sem0), (rows1, wbuf1, sem1))

    def issue(g, j, rows, wbuf, sem):
        e0 = sid * EPT + g * (GC * K) + j * K
        pltpu.async_copy(h1_hbm.at[srcbuf.at[j]], rows, sem)
        pltpu.async_copy(we_hbm.at[pl.ds(e0, K)], wbuf, sem)

    def drain(rows, wbuf, sem):
        pltpu.make_async_copy(we_hbm.at[pl.ds(0, K)], rows, sem).wait()
        pltpu.make_async_copy(we_hbm.at[pl.ds(0, K)], wbuf, sem).wait()

    def group(g, carry):
        pltpu.sync_copy(src4.at[sid, g], srcbuf)
        pltpu.sync_copy(dst4.at[sid, g], dstbuf)
        issue(g, 0, *rw[0])
        for j in range(GC):
            rows, wbuf, sem = rw[j % 2]
            nrows, nwbuf, nsem = rw[(j + 1) % 2]
            if j + 1 < GC:
                issue(g, j + 1, nrows, nwbuf, nsem)
            drain(rows, wbuf, sem)
            _mul_rows(rows, wbuf)
            pltpu.sync_copy(rows, agg_s.at[dstbuf.at[j]], add=True)
        return carry

    lax.fori_loop(0, NG, group, 0)
    plsc.subcore_barrier()

    for z in range(NPT // ZR):
        o = sid * NPT + z * ZR
        pltpu.sync_copy(agg_s.at[pl.ds(o, ZR)], out_hbm.at[pl.ds(o, ZR)])


def _sc_body(h1a, h1b, wea, web, src4, dst4, zer, agga, aggb,
             agg_s, srcbuf, dstbuf, rows0, wbuf0, rows1, wbuf1, sem0, sem1):
    cid = lax.axis_index("c")
    sid = lax.axis_index("s")

    @pl.when(cid == 0)
    def _():
        _sc_half(h1a, wea, src4, dst4, agga, sid, agg_s, srcbuf, dstbuf,
                 rows0, wbuf0, rows1, wbuf1, zer, sem0, sem1)

    @pl.when(cid == 1)
    def _():
        _sc_half(h1b, web, src4, dst4, aggb, sid, agg_s, srcbuf, dstbuf,
                 rows0, wbuf0, rows1, wbuf1, zer, sem0, sem1)


def kernel(x, rbf, dists, edge_index, batch, W_n2m, W_r2m, W_lin1, W_lin2,
           b_lin2, W_m2n):
    f32 = jnp.float32
    sds = jax.ShapeDtypeStruct

    wc1, wc2, bvec = pl.pallas_call(
        _wfold_body,
        out_shape=(sds((HIDDEN, HIDDEN), f32), sds((HIDDEN, HIDDEN), f32),
                   sds((1, HIDDEN), f32)),
    )(W_n2m, W_lin1, W_lin2, b_lin2.reshape(1, HIDDEN), W_m2n)

    NB = 5
    h1a, h1b = pl.pallas_call(
        _h1_body,
        grid=(NB,),
        in_specs=[pl.BlockSpec((N_NODES // NB, HIDDEN), lambda i: (i, 0)),
                  pl.BlockSpec((HIDDEN, HIDDEN), lambda i: (0, 0))],
        out_specs=[pl.BlockSpec((N_NODES // NB, FH), lambda i: (i, 0)),
                   pl.BlockSpec((N_NODES // NB, FH), lambda i: (i, 0))],
        out_shape=(sds((N_NODES, FH), f32), sds((N_NODES, FH), f32)),
    )(x, wc1)

    cden = pl.pallas_call(
        _cos_body,
        out_shape=sds((N_EDGES // 128, 128), f32),
    )(dists.reshape(N_EDGES // 128, 128))

    EB = 32
    wea, web = pl.pallas_call(
        _edge_body,
        grid=(EB,),
        in_specs=[pl.BlockSpec((N_EDGES // EB, RADIAL), lambda i: (i, 0)),
                  pl.BlockSpec((N_EDGES // EB, 1), lambda i: (i, 0)),
                  pl.BlockSpec((RADIAL, HIDDEN), lambda i: (0, 0))],
        out_specs=[pl.BlockSpec((N_EDGES // EB, FH), lambda i: (i, 0)),
                   pl.BlockSpec((N_EDGES // EB, FH), lambda i: (i, 0))],
        out_shape=(sds((N_EDGES, FH), f32), sds((N_EDGES, FH), f32)),
    )(rbf, cden.reshape(N_EDGES, 1), W_r2m)

    src4 = edge_index[0].astype(jnp.int32).reshape(NS, NG, GC, K)
    dst4 = edge_index[1].astype(jnp.int32).reshape(NS, NG, GC, K)
    zer = jnp.zeros((ZR, FH), f32)

    mesh = plsc.VectorSubcoreMesh(core_axis_name="c", subcore_axis_name="s",
                                  num_cores=NC, num_subcores=NS)
    agga, aggb = pl.kernel(
        _sc_body,
        out_type=(sds((NPAD, FH), f32), sds((NPAD, FH), f32)),
        mesh=mesh,
        scratch_types=[
            pltpu.VMEM_SHARED((NPAD, FH), f32),
            pltpu.VMEM((GC, K), jnp.int32),
            pltpu.VMEM((GC, K), jnp.int32),
            pltpu.VMEM((K, FH), f32),
            pltpu.VMEM((K, FH), f32),
            pltpu.VMEM((K, FH), f32),
            pltpu.VMEM((K, FH), f32),
            pltpu.SemaphoreType.DMA,
            pltpu.SemaphoreType.DMA,
        ],
    )(h1a, h1b, wea, web, src4, dst4, zer)

    out = pl.pallas_call(
        _out_body,
        grid=(NB,),
        in_specs=[pl.BlockSpec((N_NODES // NB, FH), lambda i: (i, 0)),
                  pl.BlockSpec((N_NODES // NB, FH), lambda i: (i, 0)),
                  pl.BlockSpec((HIDDEN, HIDDEN), lambda i: (0, 0)),
                  pl.BlockSpec((1, HIDDEN), lambda i: (0, 0))],
        out_specs=pl.BlockSpec((N_NODES // NB, HIDDEN), lambda i: (i, 0)),
        out_shape=sds((N_NODES, HIDDEN), f32),
    )(agga, aggb, wc2, bvec)
    return out


# speedup vs baseline: 2.5581x; 1.0232x over previous
import functools

import numpy as np

import jax
import jax.numpy as jnp
from jax import lax
from jax.experimental import pallas as pl
from jax.experimental.pallas import tpu as pltpu
from jax.experimental.pallas import tpu_sc as plsc

N_NODES = 10000
N_EDGES = 160000
HIDDEN = 256
RADIAL = 64
FH = 128
FW = 64
CUTOFF = 5.0

NC = 2
NS = 16
EPT = N_EDGES // NS
K = 40
NCHUNK = EPT // K
NG = 25
GC = NCHUNK // NG
NPAD = 10240
NPT = NPAD // NS
ZR = 128


def _wfold_body(wn2m, wlin1, wlin2, b2, wm2n, wc1, wc2, bvec):
    f32 = jnp.float32
    wc1[...] = jnp.dot(wn2m[...], wlin1[...], preferred_element_type=f32)
    wc2[...] = jnp.dot(wlin2[...], wm2n[...], preferred_element_type=f32)
    bvec[...] = jnp.dot(b2[...], wm2n[...], preferred_element_type=f32)


def _h1_body(x, wc1, h1a, h1b):
    h = jnp.dot(x[...], wc1[...],
                preferred_element_type=jnp.float32)
    h1a[...] = h[:, :FH]
    h1b[...] = h[:, FH:]


def _cos_body(d2, c2):
    c2[...] = 0.5 * (jnp.cos(d2[...] * (jnp.pi / CUTOFF)) + 1.0)


def _pack_words(lo, hi):
    ai = lax.bitcast_convert_type(lo.astype(jnp.bfloat16), jnp.uint16)
    bi = lax.bitcast_convert_type(hi.astype(jnp.bfloat16), jnp.uint16)
    return ai.astype(jnp.int32) | (bi.astype(jnp.int32) << 16)


def _edge_body(rbf, c, wr2m, wea, web):
    w = jnp.dot(rbf[...] * c[...], wr2m[...],
                preferred_element_type=jnp.float32)
    wea[...] = _pack_words(w[:, 0 * FW:1 * FW], w[:, 1 * FW:2 * FW])
    web[...] = _pack_words(w[:, 2 * FW:3 * FW], w[:, 3 * FW:4 * FW])


def _out_body(agga, aggb, wc2, bvec, out):
    acc = jnp.dot(agga[...], wc2[:FH, :], preferred_element_type=jnp.float32)
    acc += jnp.dot(aggb[...], wc2[FH:, :], preferred_element_type=jnp.float32)
    out[...] = acc + bvec[...]


def _mul_rows(rows, wbuf):
    @plsc.parallel_loop(0, K, 1, unroll=4)
    def _(r):
        for cc in range(FH // 32):
            wi = wbuf[r, pl.ds(cc * 16, 16)]
            lo = lax.bitcast_convert_type(wi << 16, jnp.float32)
            hi = lax.bitcast_convert_type(wi & jnp.int32(-65536), jnp.float32)
            s0 = pl.ds(cc * 32, 16)
            s1 = pl.ds(cc * 32 + 16, 16)
            rows[r, s0] = rows[r, s0] * lo
            rows[r, s1] = rows[r, s1] * hi


def _sc_half(h1_hbm, we_hbm, src4, dst4, out_hbm, sid, agg_s, srcbuf, dstbuf,
             rows0, wbuf0, rows1, wbuf1, zer_hbm, sem0, sem1):
    for z in range(NPT // ZR):
        pltpu.sync_copy(zer_hbm, agg_s.at[pl.ds(sid * NPT + z * ZR, ZR)])
    plsc.subcore_barrier()

    rw = ((rows0, wbuf0, sem0), (rows1, wbuf1, sem1))

    def issue(g, j, rows, wbuf, sem):
        e0 = sid * EPT + g * (GC * K) + j * K
        pltpu.async_copy(h1_hbm.at[srcbuf.at[j]], rows, sem)
        pltpu.async_copy(we_hbm.at[pl.ds(e0, K)], wbuf, sem)

    def drain(rows, wbuf, sem):
        pltpu.make_async_copy(h1_hbm.at[pl.ds(0, K)], rows, sem).wait()
        pltpu.make_async_copy(we_hbm.at[pl.ds(0, K)], wbuf, sem).wait()

    def group(g, carry):
        pltpu.sync_copy(src4.at[sid, g], srcbuf)
        pltpu.sync_copy(dst4.at[sid, g], dstbuf)
        issue(g, 0, *rw[0])
        for j in range(GC):
            rows, wbuf, sem = rw[j % 2]
            nrows, nwbuf, nsem = rw[(j + 1) % 2]
            if j + 1 < GC:
                issue(g, j + 1, nrows, nwbuf, nsem)
            drain(rows, wbuf, sem)
            _mul_rows(rows, wbuf)
            pltpu.sync_copy(rows, agg_s.at[dstbuf.at[j]], add=True)
        return carry

    lax.fori_loop(0, NG, group, 0)
    plsc.subcore_barrier()

    for z in range(NPT // ZR):
        o = sid * NPT + z * ZR
        pltpu.sync_copy(agg_s.at[pl.ds(o, ZR)], out_hbm.at[pl.ds(o, ZR)])


def _sc_body(h1a, h1b, wea, web, src4, dst4, zer, agga, aggb,
             agg_s, srcbuf, dstbuf, rows0, wbuf0, rows1, wbuf1, sem0, sem1):
    cid = lax.axis_index("c")
    sid = lax.axis_index("s")

    @pl.when(cid == 0)
    def _():
        _sc_half(h1a, wea, src4, dst4, agga, sid, agg_s, srcbuf, dstbuf,
                 rows0, wbuf0, rows1, wbuf1, zer, sem0, sem1)

    @pl.when(cid == 1)
    def _():
        _sc_half(h1b, web, src4, dst4, aggb, sid, agg_s, srcbuf, dstbuf,
                 rows0, wbuf0, rows1, wbuf1, zer, sem0, sem1)


def kernel(x, rbf, dists, edge_index, batch, W_n2m, W_r2m, W_lin1, W_lin2,
           b_lin2, W_m2n):
    f32 = jnp.float32
    sds = jax.ShapeDtypeStruct

    wc1, wc2, bvec = pl.pallas_call(
        _wfold_body,
        out_shape=(sds((HIDDEN, HIDDEN), f32), sds((HIDDEN, HIDDEN), f32),
                   sds((1, HIDDEN), f32)),
    )(W_n2m, W_lin1, W_lin2, b_lin2.reshape(1, HIDDEN), W_m2n)

    NB = 5
    h1a, h1b = pl.pallas_call(
        _h1_body,
        grid=(NB,),
        in_specs=[pl.BlockSpec((N_NODES // NB, HIDDEN), lambda i: (i, 0)),
                  pl.BlockSpec((HIDDEN, HIDDEN), lambda i: (0, 0))],
        out_specs=[pl.BlockSpec((N_NODES // NB, FH), lambda i: (i, 0)),
                   pl.BlockSpec((N_NODES // NB, FH), lambda i: (i, 0))],
        out_shape=(sds((N_NODES, FH), f32), sds((N_NODES, FH), f32)),
    )(x, wc1)

    cden = pl.pallas_call(
        _cos_body,
        out_shape=sds((N_EDGES // 128, 128), f32),
    )(dists.reshape(N_EDGES // 128, 128))

    L = np.arange(FW)
    plo = (L // 16) * 32 + L % 16
    wperm = np.concatenate([plo, plo + 16, plo + FH, plo + FH + 16])
    W_r2m_p = W_r2m[:, wperm]

    EB = 32
    wea, web = pl.pallas_call(
        _edge_body,
        grid=(EB,),
        in_specs=[pl.BlockSpec((N_EDGES // EB, RADIAL), lambda i: (i, 0)),
                  pl.BlockSpec((N_EDGES // EB, 1), lambda i: (i, 0)),
                  pl.BlockSpec((RADIAL, HIDDEN), lambda i: (0, 0))],
        out_specs=[pl.BlockSpec((N_EDGES // EB, FW), lambda i: (i, 0)),
                   pl.BlockSpec((N_EDGES // EB, FW), lambda i: (i, 0))],
        out_shape=(sds((N_EDGES, FW), jnp.int32),
                   sds((N_EDGES, FW), jnp.int32)),
    )(rbf, cden.reshape(N_EDGES, 1), W_r2m_p)

    src4 = edge_index[0].astype(jnp.int32).reshape(NS, NG, GC, K)
    dst4 = edge_index[1].astype(jnp.int32).reshape(NS, NG, GC, K)
    zer = jnp.zeros((ZR, FH), f32)

    mesh = plsc.VectorSubcoreMesh(core_axis_name="c", subcore_axis_name="s",
                                  num_cores=NC, num_subcores=NS)
    agga, aggb = pl.kernel(
        _sc_body,
        out_type=(sds((NPAD, FH), f32), sds((NPAD, FH), f32)),
        mesh=mesh,
        scratch_types=[
            pltpu.VMEM_SHARED((NPAD, FH), f32),
            pltpu.VMEM((GC, K), jnp.int32),
            pltpu.VMEM((GC, K), jnp.int32),
            pltpu.VMEM((K, FH), f32),
            pltpu.VMEM((K, FW), jnp.int32),
            pltpu.VMEM((K, FH), f32),
            pltpu.VMEM((K, FW), jnp.int32),
            pltpu.SemaphoreType.DMA,
            pltpu.SemaphoreType.DMA,
        ],
    )(h1a, h1b, wea, web, src4, dst4, zer)

    out = pl.pallas_call(
        _out_body,
        grid=(NB,),
        in_specs=[pl.BlockSpec((N_NODES // NB, FH), lambda i: (i, 0)),
                  pl.BlockSpec((N_NODES // NB, FH), lambda i: (i, 0)),
                  pl.BlockSpec((HIDDEN, HIDDEN), lambda i: (0, 0)),
                  pl.BlockSpec((1, HIDDEN), lambda i: (0, 0))],
        out_specs=pl.BlockSpec((N_NODES // NB, HIDDEN), lambda i: (i, 0)),
        out_shape=sds((N_NODES, HIDDEN), f32),
    )(agga, aggb, wc2, bvec)
    return out
